# baseline (speedup 1.0000x reference)
import jax
import jax.numpy as jnp
from jax.experimental import pallas as pl
from jax.experimental.pallas import tpu as pltpu

_BM = 5000


def _mm_kernel(x_ref, w_ref, b_ref, o_ref):
    w = w_ref[...]
    bias = b_ref[...]
    o_ref[0] = jnp.dot(x_ref[0], w, preferred_element_type=jnp.float32) + bias
    o_ref[1] = jnp.dot(x_ref[1], w, preferred_element_type=jnp.float32) + bias


def kernel(input, W, b):
    n, d = input.shape
    h = n // 2
    x3 = input.reshape(2, h, d)
    b2 = b.reshape(1, d)
    grid = (h // _BM,)
    y3 = pl.pallas_call(
        _mm_kernel,
        grid=grid,
        in_specs=[
            pl.BlockSpec((2, _BM, d), lambda i: (0, i, 0)),
            pl.BlockSpec((d, d), lambda i: (0, 0)),
            pl.BlockSpec((1, d), lambda i: (0, 0)),
        ],
        out_specs=pl.BlockSpec((2, _BM, d), lambda i: (0, i, 0)),
        out_shape=jax.ShapeDtypeStruct((2, h, d), jnp.float32),
        compiler_params=pltpu.CompilerParams(
            dimension_semantics=("parallel",),
            vmem_limit_bytes=128 * 1024 * 1024,
        ),
    )(x3, W, b2)
    return y3.reshape(n, d)
